# Initial kernel scaffold; baseline (speedup 1.0000x reference)
import jax
import jax.numpy as jnp
from jax.experimental import pallas as pl


def kernel(input, adj, W, a):
    raise NotImplementedError("write your pallas kernel here")


# speedup vs baseline: 16.8398x; 16.8398x over previous
import functools

import jax
import jax.numpy as jnp
from jax import lax
from jax.experimental import pallas as pl
from jax.experimental.pallas import tpu as pltpu
from jax.experimental.pallas import tpu_sc as plsc

N = 10000
E = 320000
D = 128
ALPHA = 0.2
NPAD = 10240
NC, NS = 2, 16
NW = NC * NS
EPW = E // NW
CHUNK = 80
NCHUNK = EPW // CHUNK
ROWS_A = 624
DSLICE = NPAD // NS


def _tc_head(blk, x_ref, w_ref, a8_ref, h_ref, s_ref):
    i = pl.program_id(0)
    h = jnp.dot(x_ref[:], w_ref[:], preferred_element_type=jnp.float32)
    h_ref[:] = h
    s8 = lax.dot_general(a8_ref[:], h, (((1,), (1,)), ((), ())),
                         preferred_element_type=jnp.float32)
    s_ref[pl.ds(i * blk, blk)] = s8[0, :]
    s_ref[pl.ds(NPAD + i * blk, blk)] = s8[1, :]


def _tc_tail(blk, u0_ref, u1_ref, d_ref, o_ref):
    i = pl.program_id(0)
    u = u0_ref[:] + u1_ref[:]
    dd = (d_ref[0, pl.ds(i * blk, blk)] + d_ref[1, pl.ds(i * blk, blk)])
    ddc = dd[:, None]
    r = u / ddc
    elu = jnp.where(r > 0, r, jnp.exp(r) - 1.0)
    o_ref[:] = jnp.where(ddc > 0, elu, 0.0)


def _sc_edges(h_hbm, s_hbm, src_hbm, dst_hbm, u_hbm, d_hbm,
              s1_v, s2_v, sv, dv, rows, ee, zd, u_sp, d_sp, sem):
    cid = lax.axis_index("c")
    sid = lax.axis_index("s")
    wid = cid * NS + sid
    base_e = wid * EPW

    pltpu.sync_copy(s_hbm.at[pl.ds(0, NPAD)], s1_v)
    pltpu.sync_copy(s_hbm.at[pl.ds(NPAD, NPAD)], s2_v)

    zv = jnp.zeros((16,), jnp.float32)

    def _zero_rows(i, _):
        for r in range(8):
            rows[i, pl.ds(r * 16, 16)] = zv
        return _
    lax.fori_loop(0, CHUNK, _zero_rows, None)

    def _zero_zd(i, _):
        zd[pl.ds(i * 16, 16)] = zv
        return _
    lax.fori_loop(0, DSLICE // 16, _zero_zd, None)

    for k in range(7):
        pltpu.sync_copy(rows, u_sp.at[pl.ds(sid * ROWS_A + k * 80, 80)])
    pltpu.sync_copy(rows.at[pl.ds(0, 64)],
                    u_sp.at[pl.ds(sid * ROWS_A + 560, 64)])

    @pl.when(sid == NS - 1)
    def _zero_tail():
        pltpu.sync_copy(rows.at[pl.ds(0, 16)], u_sp.at[pl.ds(N - 16, 16)])

    pltpu.sync_copy(zd, d_sp.at[pl.ds(sid * DSLICE, DSLICE)])
    plsc.subcore_barrier()

    def _chunk(c, _):
        off = base_e + c * CHUNK
        pltpu.sync_copy(src_hbm.at[pl.ds(off, CHUNK)], sv)
        pltpu.sync_copy(dst_hbm.at[pl.ds(off, CHUNK)], dv)
        pltpu.async_copy(h_hbm.at[dv], rows, sem).wait()
        for i in range(CHUNK // 16):
            sv16 = sv[pl.ds(i * 16, 16)]
            dv16 = dv[pl.ds(i * 16, 16)]
            e = (plsc.load_gather(s1_v, [sv16])
                 + plsc.load_gather(s2_v, [dv16]))
            e = jnp.where(e >= 0, e, ALPHA * e)
            ee[pl.ds(i * 16, 16)] = jnp.exp(e)

        def _scale(g, _):
            w16 = ee[pl.ds(g * 16, 16)]
            base = g * 16
            for j in range(16):
                w = w16[j]
                for r in range(8):
                    rows[base + j, pl.ds(r * 16, 16)] = (
                        rows[base + j, pl.ds(r * 16, 16)] * w)
            return _
        lax.fori_loop(0, CHUNK // 16, _scale, None)

        pltpu.sync_copy(rows, u_sp.at[sv], add=True)
        pltpu.sync_copy(ee, d_sp.at[sv], add=True)
        return _

    lax.fori_loop(0, NCHUNK, _chunk, None)
    plsc.subcore_barrier()

    for k in range(7):
        r0 = sid * ROWS_A + k * 80
        pltpu.sync_copy(u_sp.at[pl.ds(r0, 80)], rows)
        pltpu.sync_copy(rows, u_hbm.at[cid, pl.ds(r0, 80)])
    r0 = sid * ROWS_A + 560
    pltpu.sync_copy(u_sp.at[pl.ds(r0, 64)], rows.at[pl.ds(0, 64)])
    pltpu.sync_copy(rows.at[pl.ds(0, 64)], u_hbm.at[cid, pl.ds(r0, 64)])

    @pl.when(sid == NS - 1)
    def _flush_tail():
        pltpu.sync_copy(u_sp.at[pl.ds(N - 16, 16)], rows.at[pl.ds(0, 16)])
        pltpu.sync_copy(rows.at[pl.ds(0, 16)], u_hbm.at[cid, pl.ds(N - 16, 16)])

    pltpu.sync_copy(d_sp.at[pl.ds(sid * DSLICE, DSLICE)], zd)
    pltpu.sync_copy(zd, d_hbm.at[pl.ds(cid * NPAD + sid * DSLICE, DSLICE)])


@jax.jit
def kernel(input, adj, W, a):
    x = input
    src = adj[0]
    dst = adj[1]
    a8 = jnp.zeros((8, D), jnp.float32)
    a8 = a8.at[0].set(a[:D, 0]).at[1].set(a[D:, 0])

    blk = 2048
    h, s = pl.pallas_call(
        functools.partial(_tc_head, blk),
        grid=(pl.cdiv(N, blk),),
        in_specs=[
            pl.BlockSpec((blk, D), lambda i: (i, 0)),
            pl.BlockSpec((D, D), lambda i: (0, 0)),
            pl.BlockSpec((8, D), lambda i: (0, 0)),
        ],
        out_specs=[
            pl.BlockSpec((blk, D), lambda i: (i, 0)),
            pl.BlockSpec((2 * NPAD,), lambda i: (0,)),
        ],
        out_shape=[
            jax.ShapeDtypeStruct((N, D), jnp.float32),
            jax.ShapeDtypeStruct((2 * NPAD,), jnp.float32),
        ],
    )(x, W, a8)

    mesh = plsc.VectorSubcoreMesh(core_axis_name="c", subcore_axis_name="s",
                                  num_cores=NC, num_subcores=NS)
    sc = pl.kernel(
        _sc_edges,
        out_type=(
            jax.ShapeDtypeStruct((NC, N, D), jnp.float32),
            jax.ShapeDtypeStruct((NC * NPAD,), jnp.float32),
        ),
        mesh=mesh,
        compiler_params=pltpu.CompilerParams(needs_layout_passes=False),
        scratch_types=[
            pltpu.VMEM((NPAD,), jnp.float32),
            pltpu.VMEM((NPAD,), jnp.float32),
            pltpu.VMEM((CHUNK,), jnp.int32),
            pltpu.VMEM((CHUNK,), jnp.int32),
            pltpu.VMEM((CHUNK, D), jnp.float32),
            pltpu.VMEM((CHUNK,), jnp.float32),
            pltpu.VMEM((DSLICE,), jnp.float32),
            pltpu.VMEM_SHARED((N, D), jnp.float32),
            pltpu.VMEM_SHARED((NPAD,), jnp.float32),
            pltpu.SemaphoreType.DMA,
        ],
    )
    u2, d2 = sc(h, s, src, dst)
    d2 = d2.reshape(NC, NPAD)

    blk3 = 2048
    out = pl.pallas_call(
        functools.partial(_tc_tail, blk3),
        grid=(pl.cdiv(N, blk3),),
        in_specs=[
            pl.BlockSpec((blk3, D), lambda i: (i, 0)),
            pl.BlockSpec((blk3, D), lambda i: (i, 0)),
            pl.BlockSpec((NC, NPAD), lambda i: (0, 0)),
        ],
        out_specs=pl.BlockSpec((blk3, D), lambda i: (i, 0)),
        out_shape=jax.ShapeDtypeStruct((N, D), jnp.float32),
    )(u2[0], u2[1], d2)
    return out


# speedup vs baseline: 25.9062x; 1.5384x over previous
import functools

import jax
import jax.numpy as jnp
from jax import lax
from jax.experimental import pallas as pl
from jax.experimental.pallas import tpu as pltpu
from jax.experimental.pallas import tpu_sc as plsc

N = 10000
E = 320000
D = 128
ALPHA = 0.2
NPAD = 10240
NC, NS = 2, 16
NW = NC * NS
EPW = E // NW
CHUNK = 80
NCHUNK = EPW // CHUNK
ROWS_A = 624
DSLICE = NPAD // NS


def _tc_head(blk, x_ref, w_ref, a8_ref, h_ref, s_ref):
    i = pl.program_id(0)
    h = jnp.dot(x_ref[:], w_ref[:], preferred_element_type=jnp.float32)
    h_ref[:] = h
    s8 = lax.dot_general(a8_ref[:], h, (((1,), (1,)), ((), ())),
                         preferred_element_type=jnp.float32)
    s_ref[pl.ds(i * blk, blk)] = s8[0, :]
    s_ref[pl.ds(NPAD + i * blk, blk)] = s8[1, :]


def _tc_tail(blk, u0_ref, u1_ref, d_ref, o_ref):
    i = pl.program_id(0)
    u = u0_ref[:] + u1_ref[:]
    dd = (d_ref[0, pl.ds(i * blk, blk)] + d_ref[1, pl.ds(i * blk, blk)])
    ddc = dd[:, None]
    r = u / ddc
    elu = jnp.where(r > 0, r, jnp.exp(r) - 1.0)
    o_ref[:] = jnp.where(ddc > 0, elu, 0.0)


def _sc_edges(h_hbm, s_hbm, src_hbm, dst_hbm, u_hbm, d_hbm,
              s1_v, s2_v, ibuf, sbuf, rows, ee, zd, u_sp, d_sp,
              gsem0, gsem1, ssem0, ssem1, isem0, isem1):
    cid = lax.axis_index("c")
    sid = lax.axis_index("s")
    wid = cid * NS + sid
    base_e = wid * EPW
    gsem = (gsem0, gsem1)
    ssem = (ssem0, ssem1)
    isem = (isem0, isem1)

    pltpu.sync_copy(s_hbm.at[pl.ds(0, NPAD)], s1_v)
    pltpu.sync_copy(s_hbm.at[pl.ds(NPAD, NPAD)], s2_v)

    zv = jnp.zeros((16,), jnp.float32)

    def _zero_rows(i, _):
        for r in range(8):
            rows[0, i, pl.ds(r * 16, 16)] = zv
        return _
    lax.fori_loop(0, CHUNK, _zero_rows, None)

    def _zero_zd(i, _):
        zd[pl.ds(i * 16, 16)] = zv
        return _
    lax.fori_loop(0, DSLICE // 16, _zero_zd, None)

    for k in range(7):
        pltpu.sync_copy(rows.at[0], u_sp.at[pl.ds(sid * ROWS_A + k * 80, 80)])
    pltpu.sync_copy(rows.at[0, pl.ds(0, 64)],
                    u_sp.at[pl.ds(sid * ROWS_A + 560, 64)])

    @pl.when(sid == NS - 1)
    def _zero_tail():
        pltpu.sync_copy(rows.at[0, pl.ds(0, 16)], u_sp.at[pl.ds(N - 16, 16)])

    pltpu.sync_copy(zd, d_sp.at[pl.ds(sid * DSLICE, DSLICE)])
    plsc.subcore_barrier()

    def _coff(c):
        return base_e + jnp.minimum(c, NCHUNK - 1) * CHUNK

    def _issue_idx(c, b):
        off = _coff(c)
        pltpu.async_copy(src_hbm.at[pl.ds(off, CHUNK)], ibuf.at[b, 0], isem[b])
        pltpu.async_copy(dst_hbm.at[pl.ds(off, CHUNK)], ibuf.at[b, 1], isem[b])

    def _wait_idx(c, b):
        off = _coff(c)
        pltpu.make_async_copy(src_hbm.at[pl.ds(off, CHUNK)], ibuf.at[b, 0],
                              isem[b]).wait()
        pltpu.make_async_copy(dst_hbm.at[pl.ds(off, CHUNK)], ibuf.at[b, 1],
                              isem[b]).wait()

    def _issue_gather(b):
        pltpu.async_copy(h_hbm.at[ibuf.at[b, 1]], rows.at[b], gsem[b])

    def _wait_gather(b):
        pltpu.make_async_copy(h_hbm.at[ibuf.at[b, 1]], rows.at[b],
                              gsem[b]).wait()

    def _issue_scatter(b):
        pltpu.async_copy(rows.at[b], u_sp.at[sbuf.at[b]], ssem[b], add=True)

    def _wait_scatter(b):
        pltpu.make_async_copy(rows.at[b], u_sp.at[sbuf.at[b]], ssem[b]).wait()

    def _compute(b):
        for i in range(CHUNK // 16):
            sv16 = ibuf[b, 0, pl.ds(i * 16, 16)]
            dv16 = ibuf[b, 1, pl.ds(i * 16, 16)]
            e = (plsc.load_gather(s1_v, [sv16])
                 + plsc.load_gather(s2_v, [dv16]))
            e = jnp.where(e >= 0, e, ALPHA * e)
            ee[pl.ds(i * 16, 16)] = jnp.exp(e)
            sbuf[b, pl.ds(i * 16, 16)] = sv16
        pltpu.sync_copy(ee, d_sp.at[ibuf.at[b, 0]], add=True)

        def _scale(g, _):
            w16 = ee[pl.ds(g * 16, 16)]
            base = g * 16
            for j in range(16):
                w = w16[j]
                for r in range(8):
                    rows[b, base + j, pl.ds(r * 16, 16)] = (
                        rows[b, base + j, pl.ds(r * 16, 16)] * w)
            return _
        lax.fori_loop(0, CHUNK // 16, _scale, None)

    _issue_idx(0, 0)
    _wait_idx(0, 0)
    _issue_gather(0)
    _wait_gather(0)
    _compute(0)
    _issue_scatter(0)
    _issue_idx(1, 1)
    _wait_idx(1, 1)
    _issue_idx(2, 0)
    _issue_gather(1)

    def _pair(t, _):
        ca = 2 * t + 1
        cb = 2 * t + 2
        _wait_gather(1)
        _compute(1)
        _issue_idx(ca + 2, 1)
        _wait_scatter(0)
        _wait_idx(cb, 0)
        _issue_gather(0)
        _issue_scatter(1)
        _wait_gather(0)
        _compute(0)
        _issue_idx(cb + 2, 0)
        _wait_scatter(1)
        _wait_idx(ca + 2, 1)
        _issue_gather(1)
        _issue_scatter(0)
        return _

    lax.fori_loop(0, (NCHUNK - 1) // 2, _pair, None)

    _wait_gather(1)
    _wait_idx(NCHUNK + 1, 0)
    _wait_scatter(0)
    plsc.subcore_barrier()

    for k in range(7):
        r0 = sid * ROWS_A + k * 80
        pltpu.sync_copy(u_sp.at[pl.ds(r0, 80)], rows.at[0])
        pltpu.sync_copy(rows.at[0], u_hbm.at[cid, pl.ds(r0, 80)])
    r0 = sid * ROWS_A + 560
    pltpu.sync_copy(u_sp.at[pl.ds(r0, 64)], rows.at[0, pl.ds(0, 64)])
    pltpu.sync_copy(rows.at[0, pl.ds(0, 64)], u_hbm.at[cid, pl.ds(r0, 64)])

    @pl.when(sid == NS - 1)
    def _flush_tail():
        pltpu.sync_copy(u_sp.at[pl.ds(N - 16, 16)], rows.at[0, pl.ds(0, 16)])
        pltpu.sync_copy(rows.at[0, pl.ds(0, 16)],
                        u_hbm.at[cid, pl.ds(N - 16, 16)])

    pltpu.sync_copy(d_sp.at[pl.ds(sid * DSLICE, DSLICE)], zd)
    pltpu.sync_copy(zd, d_hbm.at[pl.ds(cid * NPAD + sid * DSLICE, DSLICE)])


@jax.jit
def kernel(input, adj, W, a):
    x = input
    src = adj[0]
    dst = adj[1]
    a8 = jnp.zeros((8, D), jnp.float32)
    a8 = a8.at[0].set(a[:D, 0]).at[1].set(a[D:, 0])

    blk = 2048
    h, s = pl.pallas_call(
        functools.partial(_tc_head, blk),
        grid=(pl.cdiv(N, blk),),
        in_specs=[
            pl.BlockSpec((blk, D), lambda i: (i, 0)),
            pl.BlockSpec((D, D), lambda i: (0, 0)),
            pl.BlockSpec((8, D), lambda i: (0, 0)),
        ],
        out_specs=[
            pl.BlockSpec((blk, D), lambda i: (i, 0)),
            pl.BlockSpec((2 * NPAD,), lambda i: (0,)),
        ],
        out_shape=[
            jax.ShapeDtypeStruct((N, D), jnp.float32),
            jax.ShapeDtypeStruct((2 * NPAD,), jnp.float32),
        ],
    )(x, W, a8)

    mesh = plsc.VectorSubcoreMesh(core_axis_name="c", subcore_axis_name="s",
                                  num_cores=NC, num_subcores=NS)
    sc = pl.kernel(
        _sc_edges,
        out_type=(
            jax.ShapeDtypeStruct((NC, N, D), jnp.float32),
            jax.ShapeDtypeStruct((NC * NPAD,), jnp.float32),
        ),
        mesh=mesh,
        compiler_params=pltpu.CompilerParams(needs_layout_passes=False),
        scratch_types=[
            pltpu.VMEM((NPAD,), jnp.float32),
            pltpu.VMEM((NPAD,), jnp.float32),
            pltpu.VMEM((2, 2, CHUNK), jnp.int32),
            pltpu.VMEM((2, CHUNK), jnp.int32),
            pltpu.VMEM((2, CHUNK, D), jnp.float32),
            pltpu.VMEM((CHUNK,), jnp.float32),
            pltpu.VMEM((DSLICE,), jnp.float32),
            pltpu.VMEM_SHARED((N, D), jnp.float32),
            pltpu.VMEM_SHARED((NPAD,), jnp.float32),
            pltpu.SemaphoreType.DMA,
            pltpu.SemaphoreType.DMA,
            pltpu.SemaphoreType.DMA,
            pltpu.SemaphoreType.DMA,
            pltpu.SemaphoreType.DMA,
            pltpu.SemaphoreType.DMA,
        ],
    )
    u2, d2 = sc(h, s, src, dst)
    d2 = d2.reshape(NC, NPAD)

    blk3 = 2048
    out = pl.pallas_call(
        functools.partial(_tc_tail, blk3),
        grid=(pl.cdiv(N, blk3),),
        in_specs=[
            pl.BlockSpec((blk3, D), lambda i: (i, 0)),
            pl.BlockSpec((blk3, D), lambda i: (i, 0)),
            pl.BlockSpec((NC, NPAD), lambda i: (0, 0)),
        ],
        out_specs=pl.BlockSpec((blk3, D), lambda i: (i, 0)),
        out_shape=jax.ShapeDtypeStruct((N, D), jnp.float32),
    )(u2[0], u2[1], d2)
    return out


# speedup vs baseline: 28.5963x; 1.1038x over previous
import functools

import jax
import jax.numpy as jnp
from jax import lax
from jax.experimental import pallas as pl
from jax.experimental.pallas import tpu as pltpu
from jax.experimental.pallas import tpu_sc as plsc

N = 10000
E = 320000
D = 128
ALPHA = 0.2
NPAD = 10240
NC, NS = 2, 16
NW = NC * NS
EPW = E // NW
CHUNK = 80
NCHUNK = EPW // CHUNK
ROWS_A = 624
DSLICE = NPAD // NS


def _tc_head(blk, x_ref, w_ref, a8_ref, h_ref, s_ref):
    i = pl.program_id(0)
    h = jnp.dot(x_ref[:], w_ref[:], preferred_element_type=jnp.float32)
    h_ref[:] = h
    s8 = lax.dot_general(a8_ref[:], h, (((1,), (1,)), ((), ())),
                         preferred_element_type=jnp.float32)
    s_ref[pl.ds(i * blk, blk)] = s8[0, :]
    s_ref[pl.ds(NPAD + i * blk, blk)] = s8[1, :]


def _tc_tail(blk, u0_ref, u1_ref, d_ref, o_ref):
    i = pl.program_id(0)
    u = u0_ref[0] + u1_ref[0]
    dd = (d_ref[0, pl.ds(i * blk, blk)] + d_ref[1, pl.ds(i * blk, blk)])
    ddc = dd[:, None]
    r = u / ddc
    elu = jnp.where(r > 0, r, jnp.exp(r) - 1.0)
    o_ref[:] = jnp.where(ddc > 0, elu, 0.0)


def _sc_edges(h_hbm, s_hbm, adj_hbm, u_hbm, d_hbm,
              s1_v, s2_v, ibuf, sbuf, rows, ee, zd, u_sp, d_sp,
              gsem0, gsem1, ssem0, ssem1, isem0, isem1, dsem0, dsem1):
    cid = lax.axis_index("c")
    sid = lax.axis_index("s")
    wid = cid * NS + sid
    base_e = wid * EPW
    gsem = (gsem0, gsem1)
    ssem = (ssem0, ssem1)
    isem = (isem0, isem1)
    dsem = (dsem0, dsem1)

    pltpu.sync_copy(s_hbm.at[pl.ds(0, NPAD)], s1_v)
    pltpu.sync_copy(s_hbm.at[pl.ds(NPAD, NPAD)], s2_v)

    zv = jnp.zeros((16,), jnp.float32)

    def _zero_rows(i, _):
        for r in range(8):
            rows[0, i, pl.ds(r * 16, 16)] = zv
        return _
    lax.fori_loop(0, CHUNK, _zero_rows, None)

    def _zero_zd(i, _):
        zd[pl.ds(i * 16, 16)] = zv
        return _
    lax.fori_loop(0, DSLICE // 16, _zero_zd, None)

    for k in range(7):
        pltpu.sync_copy(rows.at[0], u_sp.at[pl.ds(sid * ROWS_A + k * 80, 80)])
    pltpu.sync_copy(rows.at[0, pl.ds(0, 64)],
                    u_sp.at[pl.ds(sid * ROWS_A + 560, 64)])

    @pl.when(sid == NS - 1)
    def _zero_tail():
        pltpu.sync_copy(rows.at[0, pl.ds(0, 16)], u_sp.at[pl.ds(N - 16, 16)])

    pltpu.sync_copy(zd, d_sp.at[pl.ds(sid * DSLICE, DSLICE)])
    plsc.subcore_barrier()

    def _coff(c):
        return base_e + jnp.minimum(c, NCHUNK - 1) * CHUNK

    def _issue_idx(c, b):
        off = _coff(c)
        pltpu.async_copy(adj_hbm.at[pl.ds(off, CHUNK)], ibuf.at[b, 0], isem[b])
        pltpu.async_copy(adj_hbm.at[pl.ds(E + off, CHUNK)], ibuf.at[b, 1],
                         isem[b])

    def _wait_idx(c, b):
        off = _coff(c)
        pltpu.make_async_copy(adj_hbm.at[pl.ds(off, CHUNK)], ibuf.at[b, 0],
                              isem[b]).wait()
        pltpu.make_async_copy(adj_hbm.at[pl.ds(E + off, CHUNK)], ibuf.at[b, 1],
                              isem[b]).wait()

    def _issue_gather(b):
        pltpu.async_copy(h_hbm.at[ibuf.at[b, 1]], rows.at[b], gsem[b])

    def _wait_gather(b):
        pltpu.make_async_copy(h_hbm.at[ibuf.at[b, 1]], rows.at[b],
                              gsem[b]).wait()

    def _issue_scatter(b):
        pltpu.async_copy(rows.at[b], u_sp.at[sbuf.at[b]], ssem[b], add=True)

    def _wait_scatter(b):
        pltpu.make_async_copy(rows.at[b], u_sp.at[sbuf.at[b]], ssem[b]).wait()

    def _wait_dadd(b):
        pltpu.make_async_copy(ee.at[b], d_sp.at[sbuf.at[b]], dsem[b]).wait()

    def _compute(b):
        _wait_dadd(b)
        for i in range(CHUNK // 16):
            sv16 = ibuf[b, 0, pl.ds(i * 16, 16)]
            dv16 = ibuf[b, 1, pl.ds(i * 16, 16)]
            e = (plsc.load_gather(s1_v, [sv16])
                 + plsc.load_gather(s2_v, [dv16]))
            e = jnp.where(e >= 0, e, ALPHA * e)
            ee[b, pl.ds(i * 16, 16)] = jnp.exp(e)
            sbuf[b, pl.ds(i * 16, 16)] = sv16
        pltpu.async_copy(ee.at[b], d_sp.at[sbuf.at[b]], dsem[b], add=True)

        def _scale(g, _):
            w16 = ee[b, pl.ds(g * 16, 16)]
            base = g * 16
            for j in range(16):
                w = w16[j]
                for r in range(8):
                    rows[b, base + j, pl.ds(r * 16, 16)] = (
                        rows[b, base + j, pl.ds(r * 16, 16)] * w)
            return _
        lax.fori_loop(0, CHUNK // 16, _scale, None)

    for b in range(2):
        for i in range(CHUNK // 16):
            ee[b, pl.ds(i * 16, 16)] = zv
            sbuf[b, pl.ds(i * 16, 16)] = jnp.zeros((16,), jnp.int32)
        pltpu.async_copy(ee.at[b], d_sp.at[sbuf.at[b]], dsem[b], add=True)

    _issue_idx(0, 0)
    _wait_idx(0, 0)
    _issue_gather(0)
    _wait_gather(0)
    _compute(0)
    _issue_scatter(0)
    _issue_idx(1, 1)
    _wait_idx(1, 1)
    _issue_idx(2, 0)
    _issue_gather(1)

    def _pair(t, _):
        ca = 2 * t + 1
        cb = 2 * t + 2
        _wait_gather(1)
        _compute(1)
        _issue_idx(ca + 2, 1)
        _wait_scatter(0)
        _wait_idx(cb, 0)
        _issue_gather(0)
        _issue_scatter(1)
        _wait_gather(0)
        _compute(0)
        _issue_idx(cb + 2, 0)
        _wait_scatter(1)
        _wait_idx(ca + 2, 1)
        _issue_gather(1)
        _issue_scatter(0)
        return _

    lax.fori_loop(0, (NCHUNK - 1) // 2, _pair, None)

    _wait_gather(1)
    _wait_idx(NCHUNK + 1, 0)
    _wait_scatter(0)
    _wait_dadd(1)
    _wait_dadd(0)
    plsc.subcore_barrier()

    for k in range(7):
        r0 = sid * ROWS_A + k * 80
        pltpu.sync_copy(u_sp.at[pl.ds(r0, 80)], rows.at[0])
        pltpu.sync_copy(rows.at[0], u_hbm.at[cid, pl.ds(r0, 80)])
    r0 = sid * ROWS_A + 560
    pltpu.sync_copy(u_sp.at[pl.ds(r0, 64)], rows.at[0, pl.ds(0, 64)])
    pltpu.sync_copy(rows.at[0, pl.ds(0, 64)], u_hbm.at[cid, pl.ds(r0, 64)])

    @pl.when(sid == NS - 1)
    def _flush_tail():
        pltpu.sync_copy(u_sp.at[pl.ds(N - 16, 16)], rows.at[0, pl.ds(0, 16)])
        pltpu.sync_copy(rows.at[0, pl.ds(0, 16)],
                        u_hbm.at[cid, pl.ds(N - 16, 16)])

    pltpu.sync_copy(d_sp.at[pl.ds(sid * DSLICE, DSLICE)], zd)
    pltpu.sync_copy(zd, d_hbm.at[pl.ds(cid * NPAD + sid * DSLICE, DSLICE)])


@jax.jit
def kernel(input, adj, W, a):
    x = input
    adj_flat = adj.reshape(2 * E)
    a8 = jnp.zeros((8, D), jnp.float32)
    a8 = a8.at[0].set(a[:D, 0]).at[1].set(a[D:, 0])

    blk = 2048
    h, s = pl.pallas_call(
        functools.partial(_tc_head, blk),
        grid=(pl.cdiv(N, blk),),
        in_specs=[
            pl.BlockSpec((blk, D), lambda i: (i, 0)),
            pl.BlockSpec((D, D), lambda i: (0, 0)),
            pl.BlockSpec((8, D), lambda i: (0, 0)),
        ],
        out_specs=[
            pl.BlockSpec((blk, D), lambda i: (i, 0)),
            pl.BlockSpec((2 * NPAD,), lambda i: (0,)),
        ],
        out_shape=[
            jax.ShapeDtypeStruct((N, D), jnp.float32),
            jax.ShapeDtypeStruct((2 * NPAD,), jnp.float32),
        ],
    )(x, W, a8)

    mesh = plsc.VectorSubcoreMesh(core_axis_name="c", subcore_axis_name="s",
                                  num_cores=NC, num_subcores=NS)
    sc = pl.kernel(
        _sc_edges,
        out_type=(
            jax.ShapeDtypeStruct((NC, N, D), jnp.float32),
            jax.ShapeDtypeStruct((NC * NPAD,), jnp.float32),
        ),
        mesh=mesh,
        compiler_params=pltpu.CompilerParams(needs_layout_passes=False),
        scratch_types=[
            pltpu.VMEM((NPAD,), jnp.float32),
            pltpu.VMEM((NPAD,), jnp.float32),
            pltpu.VMEM((2, 2, CHUNK), jnp.int32),
            pltpu.VMEM((2, CHUNK), jnp.int32),
            pltpu.VMEM((2, CHUNK, D), jnp.float32),
            pltpu.VMEM((2, CHUNK), jnp.float32),
            pltpu.VMEM((DSLICE,), jnp.float32),
            pltpu.VMEM_SHARED((N, D), jnp.float32),
            pltpu.VMEM_SHARED((NPAD,), jnp.float32),
            pltpu.SemaphoreType.DMA,
            pltpu.SemaphoreType.DMA,
            pltpu.SemaphoreType.DMA,
            pltpu.SemaphoreType.DMA,
            pltpu.SemaphoreType.DMA,
            pltpu.SemaphoreType.DMA,
            pltpu.SemaphoreType.DMA,
            pltpu.SemaphoreType.DMA,
        ],
    )
    u2, d2 = sc(h, s, adj_flat)
    d2 = d2.reshape(NC, NPAD)

    blk3 = 2048
    out = pl.pallas_call(
        functools.partial(_tc_tail, blk3),
        grid=(pl.cdiv(N, blk3),),
        in_specs=[
            pl.BlockSpec((1, blk3, D), lambda i: (0, i, 0)),
            pl.BlockSpec((1, blk3, D), lambda i: (1, i, 0)),
            pl.BlockSpec((NC, NPAD), lambda i: (0, 0)),
        ],
        out_specs=pl.BlockSpec((blk3, D), lambda i: (i, 0)),
        out_shape=jax.ShapeDtypeStruct((N, D), jnp.float32),
    )(u2, u2, d2)
    return out


# speedup vs baseline: 37.8991x; 1.3253x over previous
import functools

import jax
import jax.numpy as jnp
from jax import lax
from jax.experimental import pallas as pl
from jax.experimental.pallas import tpu as pltpu
from jax.experimental.pallas import tpu_sc as plsc

N = 10000
E = 320000
D = 128
ALPHA = 0.2
NPAD = 10240
NC, NS = 2, 16
NW = NC * NS
EPW = E // NW
CHUNK = 80
NCHUNK = EPW // CHUNK
ROWS_A = 624
DSLICE = NPAD // NS


def _tc_head(blk, x_ref, w_ref, a8_ref, h_ref, s_ref):
    i = pl.program_id(0)
    h = jnp.dot(x_ref[:], w_ref[:], preferred_element_type=jnp.float32)
    h_ref[:] = h
    s8 = lax.dot_general(a8_ref[:], h, (((1,), (1,)), ((), ())),
                         preferred_element_type=jnp.float32)
    s_ref[pl.ds(i * blk, blk)] = s8[0, :]
    s_ref[pl.ds(NPAD + i * blk, blk)] = s8[1, :]


def _tc_tail(blk, u0_ref, u1_ref, d_ref, o_ref):
    i = pl.program_id(0)
    u = u0_ref[0] + u1_ref[0]
    dd = (d_ref[0, pl.ds(i * blk, blk)] + d_ref[1, pl.ds(i * blk, blk)])
    ddc = dd[:, None]
    r = u / ddc
    elu = jnp.where(r > 0, r, jnp.exp(r) - 1.0)
    o_ref[:] = jnp.where(ddc > 0, elu, 0.0)


def _sc_edges(h_hbm, s_hbm, adj_hbm, u_hbm, d_hbm,
              s1_v, s2_v, ibuf, sbuf, rows, ee, zd, u_sp, d_sp,
              gsem0, gsem1, ssem0, ssem1, isem0, isem1, dsem0, dsem1):
    cid = lax.axis_index("c")
    sid = lax.axis_index("s")
    wid = cid * NS + sid
    base_e = wid * EPW
    gsem = (gsem0, gsem1)
    ssem = (ssem0, ssem1)
    isem = (isem0, isem1)
    dsem = (dsem0, dsem1)

    pltpu.sync_copy(s_hbm.at[pl.ds(0, NPAD)], s1_v)
    pltpu.sync_copy(s_hbm.at[pl.ds(NPAD, NPAD)], s2_v)

    zv = jnp.zeros((16,), jnp.float32)

    def _zero_rows(i, _):
        for r in range(8):
            rows[0, i, pl.ds(r * 16, 16)] = zv
        return _
    lax.fori_loop(0, CHUNK, _zero_rows, None)

    def _zero_zd(i, _):
        zd[pl.ds(i * 16, 16)] = zv
        return _
    lax.fori_loop(0, DSLICE // 16, _zero_zd, None)

    for k in range(7):
        pltpu.sync_copy(rows.at[0], u_sp.at[pl.ds(sid * ROWS_A + k * 80, 80)])
    pltpu.sync_copy(rows.at[0, pl.ds(0, 64)],
                    u_sp.at[pl.ds(sid * ROWS_A + 560, 64)])

    @pl.when(sid == NS - 1)
    def _zero_tail():
        pltpu.sync_copy(rows.at[0, pl.ds(0, 16)], u_sp.at[pl.ds(N - 16, 16)])

    pltpu.sync_copy(zd, d_sp.at[pl.ds(sid * DSLICE, DSLICE)])
    plsc.subcore_barrier()

    def _coff(c):
        return base_e + jnp.minimum(c, NCHUNK - 1) * CHUNK

    def _issue_idx(c, b):
        off = _coff(c)
        pltpu.async_copy(adj_hbm.at[pl.ds(off, CHUNK)], ibuf.at[b, 0], isem[b])
        pltpu.async_copy(adj_hbm.at[pl.ds(E + off, CHUNK)], ibuf.at[b, 1],
                         isem[b])

    def _wait_idx(c, b):
        off = _coff(c)
        pltpu.make_async_copy(adj_hbm.at[pl.ds(off, CHUNK)], ibuf.at[b, 0],
                              isem[b]).wait()
        pltpu.make_async_copy(adj_hbm.at[pl.ds(E + off, CHUNK)], ibuf.at[b, 1],
                              isem[b]).wait()

    def _issue_gather(b):
        pltpu.async_copy(h_hbm.at[ibuf.at[b, 1]], rows.at[b], gsem[b])

    def _wait_gather(b):
        pltpu.make_async_copy(h_hbm.at[ibuf.at[b, 1]], rows.at[b],
                              gsem[b]).wait()

    def _issue_scatter(b):
        pltpu.async_copy(rows.at[b], u_sp.at[sbuf.at[b]], ssem[b], add=True)

    def _wait_scatter(b):
        pltpu.make_async_copy(rows.at[b], u_sp.at[sbuf.at[b]], ssem[b]).wait()

    def _wait_dadd(b):
        pltpu.make_async_copy(ee.at[b], d_sp.at[sbuf.at[b]], dsem[b]).wait()

    def _compute(b):
        _wait_dadd(b)
        for i in range(CHUNK // 16):
            sv16 = ibuf[b, 0, pl.ds(i * 16, 16)]
            dv16 = ibuf[b, 1, pl.ds(i * 16, 16)]
            e = (plsc.load_gather(s1_v, [sv16])
                 + plsc.load_gather(s2_v, [dv16]))
            e = jnp.where(e >= 0, e, ALPHA * e)
            ee[b, pl.ds(i * 16, 16)] = jnp.exp(e)
            sbuf[b, pl.ds(i * 16, 16)] = sv16
        pltpu.async_copy(ee.at[b], d_sp.at[sbuf.at[b]], dsem[b], add=True)

        def _scale(g, _):
            w16 = ee[b, pl.ds(g * 16, 16)]
            base = g * 16
            for j in range(16):
                w = w16[j]
                for r in range(8):
                    rows[b, base + j, pl.ds(r * 16, 16)] = (
                        rows[b, base + j, pl.ds(r * 16, 16)] * w)
            return _
        lax.fori_loop(0, CHUNK // 16, _scale, None)

    for b in range(2):
        for i in range(CHUNK // 16):
            ee[b, pl.ds(i * 16, 16)] = zv
            sbuf[b, pl.ds(i * 16, 16)] = jnp.zeros((16,), jnp.int32)
        pltpu.async_copy(ee.at[b], d_sp.at[sbuf.at[b]], dsem[b], add=True)

    def _zero_rows1(i, _):
        for r in range(8):
            rows[1, i, pl.ds(r * 16, 16)] = zv
        return _
    lax.fori_loop(0, CHUNK, _zero_rows1, None)
    _issue_scatter(1)

    _issue_idx(0, 0)
    _wait_idx(0, 0)
    _issue_gather(0)
    _issue_idx(1, 1)

    def _step(c, b):
        o = 1 - b
        _wait_idx(c + 1, o)
        _wait_scatter(o)
        _issue_gather(o)
        _wait_gather(b)
        _compute(b)
        _issue_idx(c + 2, b)
        _issue_scatter(b)

    def _pair(t, _):
        _step(2 * t, 0)
        _step(2 * t + 1, 1)
        return _

    lax.fori_loop(0, (NCHUNK - 1) // 2, _pair, None)

    _wait_idx(NCHUNK, 1)
    _wait_scatter(1)
    _wait_gather(0)
    _compute(0)
    _issue_scatter(0)
    _wait_scatter(0)
    _wait_dadd(1)
    _wait_dadd(0)
    plsc.subcore_barrier()

    for k in range(7):
        r0 = sid * ROWS_A + k * 80
        pltpu.sync_copy(u_sp.at[pl.ds(r0, 80)], rows.at[0])
        pltpu.sync_copy(rows.at[0], u_hbm.at[cid, pl.ds(r0, 80)])
    r0 = sid * ROWS_A + 560
    pltpu.sync_copy(u_sp.at[pl.ds(r0, 64)], rows.at[0, pl.ds(0, 64)])
    pltpu.sync_copy(rows.at[0, pl.ds(0, 64)], u_hbm.at[cid, pl.ds(r0, 64)])

    @pl.when(sid == NS - 1)
    def _flush_tail():
        pltpu.sync_copy(u_sp.at[pl.ds(N - 16, 16)], rows.at[0, pl.ds(0, 16)])
        pltpu.sync_copy(rows.at[0, pl.ds(0, 16)],
                        u_hbm.at[cid, pl.ds(N - 16, 16)])

    pltpu.sync_copy(d_sp.at[pl.ds(sid * DSLICE, DSLICE)], zd)
    pltpu.sync_copy(zd, d_hbm.at[pl.ds(cid * NPAD + sid * DSLICE, DSLICE)])


@jax.jit
def kernel(input, adj, W, a):
    x = input
    adj_flat = adj.reshape(2 * E)
    a8 = jnp.zeros((8, D), jnp.float32)
    a8 = a8.at[0].set(a[:D, 0]).at[1].set(a[D:, 0])

    blk = 2048
    h, s = pl.pallas_call(
        functools.partial(_tc_head, blk),
        grid=(pl.cdiv(N, blk),),
        in_specs=[
            pl.BlockSpec((blk, D), lambda i: (i, 0)),
            pl.BlockSpec((D, D), lambda i: (0, 0)),
            pl.BlockSpec((8, D), lambda i: (0, 0)),
        ],
        out_specs=[
            pl.BlockSpec((blk, D), lambda i: (i, 0)),
            pl.BlockSpec((2 * NPAD,), lambda i: (0,)),
        ],
        out_shape=[
            jax.ShapeDtypeStruct((N, D), jnp.float32),
            jax.ShapeDtypeStruct((2 * NPAD,), jnp.float32),
        ],
    )(x, W, a8)

    mesh = plsc.VectorSubcoreMesh(core_axis_name="c", subcore_axis_name="s",
                                  num_cores=NC, num_subcores=NS)
    sc = pl.kernel(
        _sc_edges,
        out_type=(
            jax.ShapeDtypeStruct((NC, N, D), jnp.float32),
            jax.ShapeDtypeStruct((NC * NPAD,), jnp.float32),
        ),
        mesh=mesh,
        compiler_params=pltpu.CompilerParams(needs_layout_passes=False),
        scratch_types=[
            pltpu.VMEM((NPAD,), jnp.float32),
            pltpu.VMEM((NPAD,), jnp.float32),
            pltpu.VMEM((2, 2, CHUNK), jnp.int32),
            pltpu.VMEM((2, CHUNK), jnp.int32),
            pltpu.VMEM((2, CHUNK, D), jnp.float32),
            pltpu.VMEM((2, CHUNK), jnp.float32),
            pltpu.VMEM((DSLICE,), jnp.float32),
            pltpu.VMEM_SHARED((N, D), jnp.float32),
            pltpu.VMEM_SHARED((NPAD,), jnp.float32),
            pltpu.SemaphoreType.DMA,
            pltpu.SemaphoreType.DMA,
            pltpu.SemaphoreType.DMA,
            pltpu.SemaphoreType.DMA,
            pltpu.SemaphoreType.DMA,
            pltpu.SemaphoreType.DMA,
            pltpu.SemaphoreType.DMA,
            pltpu.SemaphoreType.DMA,
        ],
    )
    u2, d2 = sc(h, s, adj_flat)
    d2 = d2.reshape(NC, NPAD)

    blk3 = 2048
    out = pl.pallas_call(
        functools.partial(_tc_tail, blk3),
        grid=(pl.cdiv(N, blk3),),
        in_specs=[
            pl.BlockSpec((1, blk3, D), lambda i: (0, i, 0)),
            pl.BlockSpec((1, blk3, D), lambda i: (1, i, 0)),
            pl.BlockSpec((NC, NPAD), lambda i: (0, 0)),
        ],
        out_specs=pl.BlockSpec((blk3, D), lambda i: (i, 0)),
        out_shape=jax.ShapeDtypeStruct((N, D), jnp.float32),
    )(u2, u2, d2)
    return out
